# baseline (speedup 1.0000x reference)
import jax
import jax.numpy as jnp
from jax import lax
from jax.experimental import pallas as pl
from jax.experimental.pallas import tpu as pltpu
from jax.experimental.pallas import tpu_sc as plsc

_B = 16384
_NC = 13
_NF = 26
_V = 100000
_D = 64
_T = 1 + _NC + _NF

_info = plsc.get_sparse_core_info()
_NCORE = _info.num_cores
_NSUB = _info.num_subcores
_NW = _NCORE * _NSUB
_ROWS_W = _B // _NW
_R = 16
_NBLK = _ROWS_W // _R


def _body(tabp_ref, xcatt_ref, xnumt_ref, w_ref, b_ref, cls_ref,
          out_ref, row_v, pair_v, idx_v, pofs_v, xcatb_v, xnumb_v,
          w_v, b_v, cls_v, gsem, osem):
    wid = lax.axis_index("s") * _NCORE + lax.axis_index("c")
    wbase = wid * _ROWS_W
    pltpu.sync_copy(w_ref, w_v)
    pltpu.sync_copy(b_ref, b_v)
    pltpu.sync_copy(cls_ref, cls_v)

    def blk_body(blk, carry):
        base = wbase + blk * _R
        qbase = pl.multiple_of(wbase + (blk // 8) * 128, 128)

        @pl.when(blk % 8 == 0)
        def _load_slabs():
            pltpu.sync_copy(xcatt_ref.at[:, pl.ds(qbase, 128)], xcatb_v)
            pltpu.sync_copy(xnumt_ref.at[:, pl.ds(qbase, 128)], xnumb_v)

        col0 = pl.multiple_of((blk % 8) * _R, _R)
        for f in range(_NF):
            v = xcatb_v[f, pl.ds(col0, _R)]
            idx_v[f, pl.ds(0, _R)] = lax.shift_right_logical(v, 1)
            pofs_v[f, pl.ds(0, _R)] = lax.shift_left(
                lax.bitwise_and(v, 1), 6)
        gathers = []
        for f in range(_NF):
            gathers.append(pltpu.async_copy(
                tabp_ref.at[f].at[idx_v.at[f]], pair_v.at[f], gsem))
        @pl.when(blk != 0)
        def _drain_out():
            pltpu.make_async_copy(
                row_v, out_ref.at[pl.ds(base, _R)], osem).wait()
        cls4 = [cls_v[pl.ds(c * 16, 16)] for c in range(_D // 16)]
        for r in range(_R):
            for c in range(_D // 16):
                row_v[r, pl.ds(c * 16, 16)] = cls4[c]
        for i in range(_NC):
            w4 = [w_v[i, pl.ds(c * 16, 16)] for c in range(_D // 16)]
            b4 = [b_v[i, pl.ds(c * 16, 16)] for c in range(_D // 16)]
            xvec = xnumb_v[i, pl.ds(col0, _R)]
            for r in range(_R):
                x = xvec[r]
                for c in range(_D // 16):
                    row_v[r, pl.ds((1 + i) * _D + c * 16, 16)] = (
                        x * w4[c] + b4[c])
        for cp in gathers:
            cp.wait()
        for f in range(_NF):
            pvec = pofs_v[f, pl.ds(0, _R)]
            for r in range(_R):
                ofs = pvec[r]
                for c in range(_D // 16):
                    row_v[r, pl.ds((1 + _NC + f) * _D + c * 16, 16)] = (
                        pair_v[f, r, pl.ds(ofs + c * 16, 16)])
        pltpu.async_copy(row_v, out_ref.at[pl.ds(base, _R)], osem)
        return carry

    lax.fori_loop(0, _NBLK, blk_body, 0)
    pltpu.make_async_copy(
        row_v, out_ref.at[pl.ds(wbase, _R)], osem).wait()


_VB = 512
_JB = _VB // 2
_NJ = (_V // 2 + _JB - 1) // _JB


def _pack_body(tabt_ref, out_ref):
    x = tabt_ref[0]
    ii = lax.broadcasted_iota(jnp.int32, (_D, _D), 0)
    jj = lax.broadcasted_iota(jnp.int32, (_D, _D), 1)
    eye = jnp.where(ii == jj, 1.0, 0.0).astype(jnp.float32)
    xt = lax.dot_general(x, eye, (((0,), (0,)), ((), ())),
                         preferred_element_type=jnp.float32)
    ri = lax.broadcasted_iota(jnp.int32, (_JB, _VB), 0)
    ci = lax.broadcasted_iota(jnp.int32, (_JB, _VB), 1)
    ev = jnp.where(ci == 2 * ri, 1.0, 0.0).astype(jnp.float32)
    od = jnp.where(ci == 2 * ri + 1, 1.0, 0.0).astype(jnp.float32)
    a = lax.dot_general(ev, xt, (((1,), (0,)), ((), ())),
                        preferred_element_type=jnp.float32)
    b = lax.dot_general(od, xt, (((1,), (0,)), ((), ())),
                        preferred_element_type=jnp.float32)
    out_ref[0] = jnp.concatenate([a, b], axis=1)


def _pack_table(cat_tables):
    tabt = cat_tables.transpose(0, 2, 1)
    return pl.pallas_call(
        _pack_body,
        grid=(_NF, _NJ),
        in_specs=[pl.BlockSpec((1, _D, _VB), lambda f, j: (f, 0, j))],
        out_specs=pl.BlockSpec((1, _JB, 2 * _D), lambda f, j: (f, j, 0)),
        out_shape=jax.ShapeDtypeStruct((_NF, _V // 2, 2 * _D), jnp.float32),
    )(tabt)


@jax.jit
def kernel(x_num, x_cat, num_W, num_b, cat_tables, cls_token):
    tabp = _pack_table(cat_tables)
    xcatt = x_cat.astype(jnp.int32).T
    xnumt = x_num.T
    cls = cls_token.reshape(_D)
    mesh = plsc.VectorSubcoreMesh(core_axis_name="c", subcore_axis_name="s")
    f = pl.kernel(
        _body,
        mesh=mesh,
        out_type=jax.ShapeDtypeStruct((_B, _T * _D), jnp.float32),
        scratch_types=[
            pltpu.VMEM((_R, _T * _D), jnp.float32),
            pltpu.VMEM((_NF, _R, 2 * _D), jnp.float32),
            pltpu.VMEM((_NF, _R), jnp.int32),
            pltpu.VMEM((_NF, _R), jnp.int32),
            pltpu.VMEM((_NF, 128), jnp.int32),
            pltpu.VMEM((_NC, 128), jnp.float32),
            pltpu.VMEM((_NC, _D), jnp.float32),
            pltpu.VMEM((_NC, _D), jnp.float32),
            pltpu.VMEM((_D,), jnp.float32),
            pltpu.SemaphoreType.DMA,
            pltpu.SemaphoreType.DMA,
        ],
    )
    return f(tabp, xcatt, xnumt, num_W, num_b, cls).reshape(_B, _T, _D)


# speedup vs baseline: 2.2199x; 2.2199x over previous
import jax
import jax.numpy as jnp
from jax import lax
from jax.experimental import pallas as pl
from jax.experimental.pallas import tpu as pltpu
from jax.experimental.pallas import tpu_sc as plsc

_B = 16384
_NC = 13
_NF = 26
_V = 100000
_D = 64
_T = 1 + _NC + _NF

_info = plsc.get_sparse_core_info()
_NCORE = _info.num_cores
_NSUB = _info.num_subcores
_NW = _NCORE * _NSUB
_ROWS_W = _B // _NW
_R = 16
_NBLK = _ROWS_W // _R


def _body(tabp_ref, xcatt_ref, xnumt_ref, w_ref, b_ref, cls_ref,
          out_ref, row_v, pair_v, idx_v, pofs_v, xcatb_v, xnumb_v,
          w_v, b_v, cls_v, gsem, osem):
    wid = lax.axis_index("s") * _NCORE + lax.axis_index("c")
    wbase = wid * _ROWS_W
    pltpu.sync_copy(w_ref, w_v)
    pltpu.sync_copy(b_ref, b_v)
    pltpu.sync_copy(cls_ref, cls_v)

    def blk_body(blk, carry):
        base = wbase + blk * _R
        qbase = pl.multiple_of(wbase + (blk // 8) * 128, 128)

        @pl.when(blk % 8 == 0)
        def _load_slabs():
            pltpu.sync_copy(xcatt_ref.at[:, pl.ds(qbase, 128)], xcatb_v)
            pltpu.sync_copy(xnumt_ref.at[:, pl.ds(qbase, 128)], xnumb_v)

        col0 = pl.multiple_of((blk % 8) * _R, _R)
        for f in range(_NF):
            v = xcatb_v[f, pl.ds(col0, _R)]
            idx_v[f, pl.ds(0, _R)] = lax.shift_right_logical(v, 1)
            pofs_v[f, pl.ds(0, _R)] = lax.shift_left(
                lax.bitwise_and(v, 1), 6)
        gathers = []
        for f in range(_NF):
            gathers.append(pltpu.async_copy(
                tabp_ref.at[f].at[idx_v.at[f]], pair_v.at[f], gsem))
        @pl.when(blk != 0)
        def _drain_out():
            pltpu.make_async_copy(
                row_v, out_ref.at[pl.ds(base, _R)], osem).wait()
        cls4 = [cls_v[pl.ds(c * 16, 16)] for c in range(_D // 16)]
        for r in range(_R):
            for c in range(_D // 16):
                row_v[r, pl.ds(c * 16, 16)] = cls4[c]
        for i in range(_NC):
            w4 = [w_v[i, pl.ds(c * 16, 16)] for c in range(_D // 16)]
            b4 = [b_v[i, pl.ds(c * 16, 16)] for c in range(_D // 16)]
            xvec = xnumb_v[i, pl.ds(col0, _R)]
            for r in range(_R):
                x = xvec[r]
                for c in range(_D // 16):
                    row_v[r, pl.ds((1 + i) * _D + c * 16, 16)] = (
                        x * w4[c] + b4[c])
        for cp in gathers:
            cp.wait()
        for f in range(_NF):
            pvec = pofs_v[f, pl.ds(0, _R)]
            for r in range(_R):
                ofs = pvec[r]
                for c in range(_D // 16):
                    row_v[r, pl.ds((1 + _NC + f) * _D + c * 16, 16)] = (
                        pair_v[f, r, pl.ds(ofs + c * 16, 16)])
        pltpu.async_copy(row_v, out_ref.at[pl.ds(base, _R)], osem)
        return carry

    lax.fori_loop(0, _NBLK, blk_body, 0)
    pltpu.make_async_copy(
        row_v, out_ref.at[pl.ds(wbase, _R)], osem).wait()


@jax.jit
def kernel(x_num, x_cat, num_W, num_b, cat_tables, cls_token):
    tabp = cat_tables.reshape(_NF, _V // 2, 2 * _D)
    xcatt = x_cat.astype(jnp.int32).T
    xnumt = x_num.T
    cls = cls_token.reshape(_D)
    mesh = plsc.VectorSubcoreMesh(core_axis_name="c", subcore_axis_name="s")
    f = pl.kernel(
        _body,
        mesh=mesh,
        out_type=jax.ShapeDtypeStruct((_B, _T * _D), jnp.float32),
        scratch_types=[
            pltpu.VMEM((_R, _T * _D), jnp.float32),
            pltpu.VMEM((_NF, _R, 2 * _D), jnp.float32),
            pltpu.VMEM((_NF, _R), jnp.int32),
            pltpu.VMEM((_NF, _R), jnp.int32),
            pltpu.VMEM((_NF, 128), jnp.int32),
            pltpu.VMEM((_NC, 128), jnp.float32),
            pltpu.VMEM((_NC, _D), jnp.float32),
            pltpu.VMEM((_NC, _D), jnp.float32),
            pltpu.VMEM((_D,), jnp.float32),
            pltpu.SemaphoreType.DMA,
            pltpu.SemaphoreType.DMA,
        ],
    )
    return f(tabp, xcatt, xnumt, num_W, num_b, cls).reshape(_B, _T, _D)
